# baseline (speedup 1.0000x reference)
import functools

import jax
import jax.numpy as jnp
from jax import lax
from jax.experimental import pallas as pl
from jax.experimental.pallas import tpu as pltpu
from jax.experimental.pallas import tpu_sc as plsc


@functools.lru_cache(maxsize=None)
def _make_sc_gather(V, D, B):
    info = plsc.get_sparse_core_info()
    NC, NS, L = info.num_cores, info.num_subcores, info.num_lanes
    NW = NC * NS
    assert D == 32 and L == 16
    PACK = 128 // D
    assert B % (NW * 4) == 0
    b_per_w = B // NW
    chunk = 320
    assert b_per_w % chunk == 0
    n_chunks = b_per_w // chunk
    sup_chunk = chunk // PACK
    mesh = plsc.VectorSubcoreMesh(core_axis_name="c", subcore_axis_name="s")

    @functools.partial(
        pl.kernel,
        mesh=mesh,
        compiler_params=pltpu.CompilerParams(
            use_tc_tiling_on_sc=True, needs_layout_passes=False
        ),
        out_type=jax.ShapeDtypeStruct((B // PACK, 128), jnp.float32),
        scratch_types=[
            pltpu.VMEM((chunk,), jnp.int32),
            pltpu.VMEM((chunk,), jnp.int32),
            pltpu.VMEM((chunk, 128), jnp.float32),
            pltpu.VMEM((sup_chunk, 128), jnp.float32),
            pltpu.SemaphoreType.DMA,
            pltpu.SemaphoreType.DMA,
        ],
    )
    def k(table_hbm, x_hbm, out_hbm, idx_v, idxs_v, rows_v, packed_v, gsem, wsem):
        wid = lax.axis_index("s") * NC + lax.axis_index("c")
        base = wid * b_per_w
        sup_base = base // PACK
        iota = lax.iota(jnp.int32, 16)

        def chunk_body(i, carry):
            off = base + i * chunk
            pltpu.sync_copy(x_hbm.at[pl.ds(off, chunk)], idx_v)
            def shift_body(g, c):
                v = idx_v[pl.ds(g * 16, 16)]
                idxs_v[pl.ds(g * 16, 16)] = lax.shift_right_logical(v, 2)
                return c

            lax.fori_loop(0, chunk // 16, shift_body, 0, unroll=True)
            pltpu.async_copy(table_hbm.at[idxs_v], rows_v, gsem).wait()

            def group_body(g, c):
                b0 = g * 16
                bvec = b0 + iota
                idxv = idx_v[pl.ds(b0, 16)]
                src_lane0 = (idxv & 3) * D
                dst_row = lax.shift_right_logical(bvec, 2)
                dst_lane0 = (bvec & 3) * D
                for d in range(D):
                    val = plsc.load_gather(rows_v, [bvec, src_lane0 + d])
                    plsc.store_scatter(packed_v, [dst_row, dst_lane0 + d], val)
                return c

            lax.fori_loop(0, chunk // 16, group_body, 0)
            sup_off = pl.multiple_of(sup_base + i * sup_chunk, 8)
            pltpu.async_copy(
                packed_v, out_hbm.at[pl.ds(sup_off, sup_chunk)], wsem
            ).wait()
            return carry

        lax.fori_loop(0, n_chunks, chunk_body, 0)

    return k


def kernel(x, table):
    S0, S1 = x.shape
    V, D = table.shape
    B = S0 * S1
    outS = _make_sc_gather(V, D, B)(
        table.reshape(V * D // 128, 128), x.reshape(B)
    )
    return outS.reshape(S0, S1, D)


# speedup vs baseline: 2.0988x; 2.0988x over previous
import functools

import jax
import jax.numpy as jnp
from jax import lax
from jax.experimental import pallas as pl
from jax.experimental.pallas import tpu as pltpu
from jax.experimental.pallas import tpu_sc as plsc


@functools.lru_cache(maxsize=None)
def _make_sc_gather(V, D, S0, S1):
    info = plsc.get_sparse_core_info()
    NC, NS = info.num_cores, info.num_subcores
    NW = NC * NS
    B = S0 * S1
    assert B % NW == 0
    b_per_w = B // NW
    chunk_rows = 32
    chunk = chunk_rows * S1
    assert b_per_w % chunk == 0
    n_chunks = b_per_w // chunk
    rows_per_w = b_per_w // S1
    nbuf = 2
    mesh = plsc.VectorSubcoreMesh(core_axis_name="c", subcore_axis_name="s")

    @functools.partial(
        pl.kernel,
        mesh=mesh,
        compiler_params=pltpu.CompilerParams(
            use_tc_tiling_on_sc=False, needs_layout_passes=False
        ),
        out_type=jax.ShapeDtypeStruct((S0, S1, D), jnp.float32),
        scratch_types=(
            [pltpu.VMEM((chunk,), jnp.int32)] * nbuf
            + [pltpu.VMEM((chunk, D), jnp.float32)] * nbuf
            + [pltpu.SemaphoreType.DMA] * (2 * nbuf)
        ),
    )
    def k(table_hbm, x_hbm, out_hbm, *scratch):
        idx_v = scratch[:nbuf]
        rows_v = scratch[nbuf : 2 * nbuf]
        gsem = scratch[2 * nbuf : 3 * nbuf]
        wsem = scratch[3 * nbuf :]
        wid = lax.axis_index("s") * NC + lax.axis_index("c")
        row_base = wid * rows_per_w

        def start_gather(i, j):
            off = (row_base + i * chunk_rows) * S1
            pltpu.sync_copy(x_hbm.at[pl.ds(off, chunk)], idx_v[j])
            return pltpu.async_copy(table_hbm.at[idx_v[j]], rows_v[j], gsem[j])

        def start_write(i, j):
            r0 = row_base + i * chunk_rows
            last = None
            for a in range(chunk_rows):
                last = pltpu.async_copy(
                    rows_v[j].at[pl.ds(a * S1, S1)],
                    out_hbm.at[r0 + a],
                    wsem[j],
                )
            return last

        def drain_write(j):
            for a in range(chunk_rows):
                w[j].wait()

        g = [None] * nbuf
        w = [None] * nbuf
        g[0] = start_gather(0, 0)
        for i in range(n_chunks):
            j = i % nbuf
            nj = (i + 1) % nbuf
            if i + 1 < n_chunks:
                if w[nj] is not None:
                    drain_write(nj)
                    w[nj] = None
                g[nj] = start_gather(i + 1, nj)
            g[j].wait()
            w[j] = start_write(i, j)
        for j in range(nbuf):
            if w[j] is not None:
                drain_write(j)

    return k


def kernel(x, table):
    S0, S1 = x.shape
    V, D = table.shape
    out = _make_sc_gather(V, D, S0, S1)(table, x.reshape(S0 * S1))
    zero = (x[0, 0] * 0).astype(out.dtype)
    return out + zero
